# baseline (speedup 1.0000x reference)
import functools
import jax
import jax.numpy as jnp
from jax import lax
from jax.experimental import pallas as pl
from jax.experimental.pallas import tpu as pltpu
from jax.experimental.pallas import tpu_sc as plsc

NUM_EMBEDDINGS = 1000000
EMBEDDING_DIM = 64
SEQ_LEN = 200
BATCH = 4096

_NC, _NS = 2, 16
_NW = _NC * _NS
_V2 = NUM_EMBEDDINGS // 2
_NB1 = (NUM_EMBEDDINGS + 127) // 128
_K1 = (_NB1 + _NW - 1) // _NW
_CB = 256
_NU = SEQ_LEN * (BATCH // _CB)
_K2 = _NU // _NW


def _iota16():
    return lax.iota(jnp.int32, 16)


def _make_retile():
    mesh = plsc.VectorSubcoreMesh(core_axis_name="c", subcore_axis_name="s")

    @functools.partial(
        pl.kernel,
        mesh=mesh,
        out_type=jax.ShapeDtypeStruct((_V2, 128), jnp.float32),
        compiler_params=pltpu.CompilerParams(
            use_tc_tiling_on_sc=True, needs_layout_passes=False),
        scratch_types=[
            pltpu.VMEM((64, 128), jnp.float32),
            pltpu.VMEM((64, 128), jnp.float32),
        ],
    )
    def retile_kernel(wt_hbm, t2_hbm, in_v, out_v):
        wid = lax.axis_index("s") * _NC + lax.axis_index("c")
        rows16 = _iota16()

        def do_transpose():
            for p in range(64):
                for h in range(2):
                    col = jnp.full((16,), 2 * p + h, jnp.int32)
                    for g in range(4):
                        vec = plsc.load_gather(in_v, [g * 16 + rows16, col])
                        out_v[p, pl.ds(h * 64 + g * 16, 16)] = vec

        def body(k, carry):
            bi = wid + k * _NW
            c0 = bi * 128

            @pl.when(bi < _NB1 - 1)
            def _full():
                pltpu.sync_copy(wt_hbm.at[:, pl.ds(c0, 128)], in_v)
                do_transpose()
                pltpu.sync_copy(out_v, t2_hbm.at[pl.ds(bi * 64, 64), :])

            @pl.when(bi == _NB1 - 1)
            def _tail():
                pltpu.sync_copy(wt_hbm.at[:, pl.ds(c0, 128)], in_v)
                do_transpose()
                pltpu.sync_copy(
                    out_v.at[pl.ds(0, 32), :],
                    t2_hbm.at[pl.ds(bi * 64, 32), :])

            return carry

        lax.fori_loop(0, _K1, body, 0)

    return retile_kernel


def _make_gather():
    mesh = plsc.VectorSubcoreMesh(core_axis_name="c", subcore_axis_name="s")

    @functools.partial(
        pl.kernel,
        mesh=mesh,
        out_type=jax.ShapeDtypeStruct((SEQ_LEN, EMBEDDING_DIM, BATCH),
                                      jnp.float32),
        compiler_params=pltpu.CompilerParams(
            use_tc_tiling_on_sc=True, needs_layout_passes=False),
        scratch_types=[
            pltpu.VMEM((_CB,), jnp.int32),
            pltpu.VMEM((_CB,), jnp.int32),
            pltpu.VMEM((_CB, 128), jnp.float32),
            pltpu.VMEM((EMBEDDING_DIM, _CB), jnp.float32),
            pltpu.SemaphoreType.DMA,
        ],
    )
    def gather_kernel(t2_hbm, idx_hbm, out_hbm, idx_v, pidx_v, g_v, o_v,
                      sem):
        wid = lax.axis_index("s") * _NC + lax.axis_index("c")
        rows16 = _iota16()

        def body(k, carry):
            u = wid + k * _NW
            t = u // (BATCH // _CB)
            b0 = (u % (BATCH // _CB)) * _CB

            pltpu.sync_copy(idx_hbm.at[t, pl.ds(b0, _CB)], idx_v)
            for ig in range(_CB // 16):
                pidx_v[pl.ds(ig * 16, 16)] = (
                    idx_v[pl.ds(ig * 16, 16)] >> 1)
            pltpu.async_copy(t2_hbm.at[pidx_v], g_v, sem).wait()
            for ig in range(_CB // 16):
                halfv = (idx_v[pl.ds(ig * 16, 16)] & 1) * 64
                rows = ig * 16 + rows16
                for d in range(EMBEDDING_DIM):
                    vec = plsc.load_gather(g_v, [rows, halfv + d])
                    o_v[d, pl.ds(ig * 16, 16)] = vec
            pltpu.sync_copy(o_v, out_hbm.at[t, :, pl.ds(b0, _CB)])
            return carry

        lax.fori_loop(0, _K2, body, 0)

    return gather_kernel


_retile = _make_retile()
_gather = _make_gather()


def kernel(input, weight):
    t2 = _retile(weight.T)
    out_t = _gather(t2, input)
    return jnp.transpose(out_t, (0, 2, 1))


# speedup vs baseline: 1.5178x; 1.5178x over previous
import functools
import jax
import jax.numpy as jnp
from jax import lax
from jax.experimental import pallas as pl
from jax.experimental.pallas import tpu as pltpu
from jax.experimental.pallas import tpu_sc as plsc

NUM_EMBEDDINGS = 1000000
EMBEDDING_DIM = 64
SEQ_LEN = 200
BATCH = 4096

_NC, _NS = 2, 16
_NW = _NC * _NS
_V2 = NUM_EMBEDDINGS // 2
_NB1 = (NUM_EMBEDDINGS + 127) // 128
_K1 = (_NB1 + _NW - 1) // _NW
_CB = 256
_NU = SEQ_LEN * (BATCH // _CB)
_K2 = _NU // _NW


def _iota16():
    return lax.iota(jnp.int32, 16)


def _make_retile():
    mesh = plsc.VectorSubcoreMesh(core_axis_name="c", subcore_axis_name="s")

    @functools.partial(
        pl.kernel,
        mesh=mesh,
        out_type=jax.ShapeDtypeStruct((_V2, 128), jnp.float32),
        compiler_params=pltpu.CompilerParams(
            use_tc_tiling_on_sc=True, needs_layout_passes=False),
        scratch_types=[
            pltpu.VMEM((64, 128), jnp.float32),
            pltpu.VMEM((64, 128), jnp.float32),
        ],
    )
    def retile_kernel(wt_hbm, t2_hbm, in_v, out_v):
        wid = lax.axis_index("s") * _NC + lax.axis_index("c")
        rows16 = _iota16()

        def do_transpose():
            @plsc.parallel_loop(0, 64, 1, unroll=4)
            def _t(p):
                for h in range(2):
                    col = jnp.full((16,), 2, jnp.int32) * p + h
                    for g in range(4):
                        vec = plsc.load_gather(in_v, [g * 16 + rows16, col])
                        out_v[p, pl.ds(h * 64 + g * 16, 16)] = vec

        def body(k, carry):
            bi = wid + k * _NW
            c0 = bi * 128

            @pl.when(bi < _NB1 - 1)
            def _full():
                pltpu.sync_copy(wt_hbm.at[:, pl.ds(c0, 128)], in_v)
                do_transpose()
                pltpu.sync_copy(out_v, t2_hbm.at[pl.ds(bi * 64, 64), :])

            @pl.when(bi == _NB1 - 1)
            def _tail():
                pltpu.sync_copy(wt_hbm.at[:, pl.ds(c0, 128)], in_v)
                do_transpose()
                pltpu.sync_copy(
                    out_v.at[pl.ds(0, 32), :],
                    t2_hbm.at[pl.ds(bi * 64, 32), :])

            return carry

        lax.fori_loop(0, _K1, body, 0)

    return retile_kernel


def _make_gather():
    mesh = plsc.VectorSubcoreMesh(core_axis_name="c", subcore_axis_name="s")

    @functools.partial(
        pl.kernel,
        mesh=mesh,
        out_type=jax.ShapeDtypeStruct((SEQ_LEN, EMBEDDING_DIM, BATCH),
                                      jnp.float32),
        compiler_params=pltpu.CompilerParams(
            use_tc_tiling_on_sc=True, needs_layout_passes=False),
        scratch_types=[
            pltpu.VMEM((_CB,), jnp.int32),
            pltpu.VMEM((_CB,), jnp.int32),
            pltpu.VMEM((_CB, 128), jnp.float32),
            pltpu.VMEM((EMBEDDING_DIM, _CB), jnp.float32),
            pltpu.SemaphoreType.DMA,
        ],
    )
    def gather_kernel(t2_hbm, idx_hbm, out_hbm, idx_v, pidx_v, g_v, o_v,
                      sem):
        wid = lax.axis_index("s") * _NC + lax.axis_index("c")
        rows16 = _iota16()

        def body(k, carry):
            u = wid + k * _NW
            t = u // (BATCH // _CB)
            b0 = (u % (BATCH // _CB)) * _CB

            pltpu.sync_copy(idx_hbm.at[t, pl.ds(b0, _CB)], idx_v)
            for ig in range(_CB // 16):
                pidx_v[pl.ds(ig * 16, 16)] = (
                    idx_v[pl.ds(ig * 16, 16)] >> 1)
            pltpu.async_copy(t2_hbm.at[pidx_v], g_v, sem).wait()
            @plsc.parallel_loop(0, _CB // 16, 1, unroll=2)
            def _t(ig):
                halfv = (idx_v[pl.ds(ig * 16, 16)] & 1) * 64
                rows = ig * 16 + rows16
                for d in range(EMBEDDING_DIM):
                    vec = plsc.load_gather(g_v, [rows, halfv + d])
                    o_v[d, pl.ds(ig * 16, 16)] = vec
            pltpu.sync_copy(o_v, out_hbm.at[t, :, pl.ds(b0, _CB)])
            return carry

        lax.fori_loop(0, _K2, body, 0)

    return gather_kernel


_retile = _make_retile()
_gather = _make_gather()


def kernel(input, weight):
    t2 = _retile(weight.T)
    out_t = _gather(t2, input)
    return jnp.transpose(out_t, (0, 2, 1))


# speedup vs baseline: 2.0805x; 1.3707x over previous
import functools
import jax
import jax.numpy as jnp
from jax import lax
from jax.experimental import pallas as pl
from jax.experimental.pallas import tpu as pltpu
from jax.experimental.pallas import tpu_sc as plsc

NUM_EMBEDDINGS = 1000000
EMBEDDING_DIM = 64
SEQ_LEN = 200
BATCH = 4096

_NC, _NS = 2, 16
_NW = _NC * _NS
_V2 = NUM_EMBEDDINGS // 2
_NB1 = (NUM_EMBEDDINGS + 127) // 128
_S1 = (_NB1 // _NW + 2) // 2
_CB = 256
_NU = SEQ_LEN * (BATCH // _CB)
_K2 = _NU // _NW

_params = pltpu.CompilerParams(
    use_tc_tiling_on_sc=True, needs_layout_passes=False)


def _iota16():
    return lax.iota(jnp.int32, 16)


def _make_retile():
    mesh = plsc.VectorSubcoreMesh(core_axis_name="c", subcore_axis_name="s")

    @functools.partial(
        pl.kernel,
        mesh=mesh,
        out_type=jax.ShapeDtypeStruct((_V2, 128), jnp.float32),
        compiler_params=_params,
        scratch_types=[
            pltpu.VMEM((2, 64, 128), jnp.float32),
            pltpu.VMEM((2, 64, 128), jnp.float32),
            pltpu.SemaphoreType.DMA,
            pltpu.SemaphoreType.DMA,
            pltpu.SemaphoreType.DMA,
            pltpu.SemaphoreType.DMA,
        ],
    )
    def retile_kernel(wt_hbm, t2_hbm, in_v, out_v, gi0, gi1, go0, go1):
        wid = lax.axis_index("s") * _NC + lax.axis_index("c")
        rows16 = _iota16()
        gis = (gi0, gi1)
        gos = (go0, go1)

        def start_in(b, bi):
            pltpu.async_copy(
                wt_hbm.at[:, pl.ds(bi * 128, 128)], in_v.at[b], gis[b])

        def wait_in(b):
            pltpu.make_async_copy(
                wt_hbm.at[:, pl.ds(0, 128)], in_v.at[b], gis[b]).wait()

        def start_out(b, bi):
            @pl.when(bi < _NB1 - 1)
            def _():
                pltpu.async_copy(
                    out_v.at[b], t2_hbm.at[pl.ds(bi * 64, 64), :], gos[b])

            @pl.when(bi == _NB1 - 1)
            def _():
                pltpu.async_copy(
                    out_v.at[b, pl.ds(0, 32)],
                    t2_hbm.at[pl.ds(bi * 64, 32), :], gos[b])

        def wait_out(b, bi):
            @pl.when(bi < _NB1 - 1)
            def _():
                pltpu.make_async_copy(
                    out_v.at[b], t2_hbm.at[pl.ds(0, 64), :], gos[b]).wait()

            @pl.when(bi == _NB1 - 1)
            def _():
                pltpu.make_async_copy(
                    out_v.at[b, pl.ds(0, 32)],
                    t2_hbm.at[pl.ds(0, 32), :], gos[b]).wait()

        def transpose(b):
            @plsc.parallel_loop(0, 64, 1, unroll=4)
            def _t(p):
                for h in range(2):
                    col = jnp.full((16,), 2, jnp.int32) * p + h
                    for g in range(4):
                        vec = plsc.load_gather(
                            in_v.at[b], [g * 16 + rows16, col])
                        out_v[b, p, pl.ds(h * 64 + g * 16, 16)] = vec

        def blk(s, k):
            return wid + (2 * s + k) * _NW

        start_in(0, blk(0, 0))

        def body(s, carry):
            bi0 = blk(s, 0)
            bi1 = blk(s, 1)
            bi2 = blk(s, 2)

            @pl.when(bi0 < _NB1)
            def _a():
                wait_in(0)

                @pl.when(bi1 < _NB1)
                def _():
                    start_in(1, bi1)

                @pl.when(s > 0)
                def _():
                    wait_out(0, bi0 - 2 * _NW)

                transpose(0)
                start_out(0, bi0)

            @pl.when(bi1 < _NB1)
            def _b():
                wait_in(1)

                @pl.when(bi2 < _NB1)
                def _():
                    start_in(0, bi2)

                @pl.when(s > 0)
                def _():
                    wait_out(1, bi1 - 2 * _NW)

                transpose(1)
                start_out(1, bi1)

            return carry

        lax.fori_loop(0, _S1, body, 0)

        last0 = wid + ((_NB1 - 1 - wid) // (2 * _NW)) * 2 * _NW
        last1 = last0 + _NW

        @pl.when(last0 < _NB1)
        def _d0():
            wait_out(0, last0)

        @pl.when(last1 < _NB1)
        def _d1():
            wait_out(1, last1)

    return retile_kernel


def _make_gather():
    mesh = plsc.VectorSubcoreMesh(core_axis_name="c", subcore_axis_name="s")

    @functools.partial(
        pl.kernel,
        mesh=mesh,
        out_type=jax.ShapeDtypeStruct((SEQ_LEN, EMBEDDING_DIM, BATCH),
                                      jnp.float32),
        compiler_params=_params,
        scratch_types=[
            pltpu.VMEM((2, _CB), jnp.int32),
            pltpu.VMEM((_CB,), jnp.int32),
            pltpu.VMEM((_CB,), jnp.int32),
            pltpu.VMEM((2, _CB, 128), jnp.float32),
            pltpu.VMEM((2, EMBEDDING_DIM, _CB), jnp.float32),
            pltpu.SemaphoreType.DMA,
            pltpu.SemaphoreType.DMA,
            pltpu.SemaphoreType.DMA,
            pltpu.SemaphoreType.DMA,
        ],
    )
    def gather_kernel(t2_hbm, idx_hbm, out_hbm, idx_v, pidx_a, pidx_b,
                      g_v, o_v, gg0, gg1, go0, go1):
        wid = lax.axis_index("s") * _NC + lax.axis_index("c")
        rows16 = _iota16()
        ggs = (gg0, gg1)
        gos = (go0, go1)
        nbt = BATCH // _CB
        pidxs = (pidx_a, pidx_b)

        def load_idx(b, u):
            t = u // nbt
            b0 = (u % nbt) * _CB
            pltpu.sync_copy(idx_hbm.at[t, pl.ds(b0, _CB)], idx_v.at[b])
            for ig in range(_CB // 16):
                pidxs[b][pl.ds(ig * 16, 16)] = (
                    idx_v[b, pl.ds(ig * 16, 16)] >> 1)

        def start_gather(b):
            pltpu.async_copy(t2_hbm.at[pidxs[b]], g_v.at[b], ggs[b])

        def wait_gather(b):
            pltpu.make_async_copy(
                t2_hbm.at[pidxs[b]], g_v.at[b], ggs[b]).wait()

        def start_out(b, u):
            t = u // nbt
            b0 = (u % nbt) * _CB
            pltpu.async_copy(
                o_v.at[b], out_hbm.at[t, :, pl.ds(b0, _CB)], gos[b])

        def wait_out(b):
            pltpu.make_async_copy(
                o_v.at[b], out_hbm.at[0, :, pl.ds(0, _CB)], gos[b]).wait()

        def transpose(b):
            @plsc.parallel_loop(0, _CB // 16, 1, unroll=2)
            def _t(ig):
                halfv = (idx_v[b, pl.ds(ig * 16, 16)] & 1) * 64
                rows = ig * 16 + rows16
                for d in range(EMBEDDING_DIM):
                    vec = plsc.load_gather(g_v.at[b], [rows, halfv + d])
                    o_v[b, d, pl.ds(ig * 16, 16)] = vec

        def unit(s, k):
            return wid + (2 * s + k) * _NW

        load_idx(0, unit(0, 0))
        start_gather(0)

        def body(s, carry):
            load_idx(1, unit(s, 1))
            wait_gather(0)
            start_gather(1)

            @pl.when(s > 0)
            def _w0():
                wait_out(0)

            transpose(0)
            start_out(0, unit(s, 0))

            @pl.when(s + 1 < _K2 // 2)
            def _li():
                load_idx(0, unit(s, 2))

            wait_gather(1)

            @pl.when(s + 1 < _K2 // 2)
            def _sg():
                start_gather(0)

            @pl.when(s > 0)
            def _w1():
                wait_out(1)

            transpose(1)
            start_out(1, unit(s, 1))
            return carry

        lax.fori_loop(0, _K2 // 2, body, 0)
        wait_out(0)
        wait_out(1)

    return gather_kernel


_retile = _make_retile()
_gather = _make_gather()


def kernel(input, weight):
    t2 = _retile(weight.T)
    out_t = _gather(t2, input)
    return jnp.transpose(out_t, (0, 2, 1))


# speedup vs baseline: 2.1335x; 1.0255x over previous
import functools
import jax
import jax.numpy as jnp
from jax import lax
from jax.experimental import pallas as pl
from jax.experimental.pallas import tpu as pltpu
from jax.experimental.pallas import tpu_sc as plsc

NUM_EMBEDDINGS = 1000000
EMBEDDING_DIM = 64
SEQ_LEN = 200
BATCH = 4096

_NC, _NS = 2, 16
_NW = _NC * _NS
_V2 = NUM_EMBEDDINGS // 2
_NB1 = (NUM_EMBEDDINGS + 127) // 128
_S1 = (_NB1 // _NW + 4) // 4
_CB = 128
_NUPW = SEQ_LEN * (BATCH // _CB) // _NW

_params = pltpu.CompilerParams(
    use_tc_tiling_on_sc=True, needs_layout_passes=False)


def _iota16():
    return lax.iota(jnp.int32, 16)


def _make_retile():
    mesh = plsc.VectorSubcoreMesh(core_axis_name="c", subcore_axis_name="s")

    @functools.partial(
        pl.kernel,
        mesh=mesh,
        out_type=jax.ShapeDtypeStruct((_V2, 128), jnp.float32),
        compiler_params=_params,
        scratch_types=[
            pltpu.VMEM((4, 64, 128), jnp.float32),
            pltpu.VMEM((2, 64, 128), jnp.float32),
            pltpu.SemaphoreType.DMA,
            pltpu.SemaphoreType.DMA,
            pltpu.SemaphoreType.DMA,
            pltpu.SemaphoreType.DMA,
            pltpu.SemaphoreType.DMA,
            pltpu.SemaphoreType.DMA,
        ],
    )
    def retile_kernel(wt_hbm, t2_hbm, in_v, out_v,
                      gi0, gi1, gi2, gi3, go0, go1):
        wid = lax.axis_index("s") * _NC + lax.axis_index("c")
        rows16 = _iota16()
        gis = (gi0, gi1, gi2, gi3)
        gos = (go0, go1)

        def start_in(j, bi):
            pltpu.async_copy(
                wt_hbm.at[:, pl.ds(bi * 128, 128)], in_v.at[j], gis[j])

        def wait_in(j):
            pltpu.make_async_copy(
                wt_hbm.at[:, pl.ds(0, 128)], in_v.at[j], gis[j]).wait()

        def start_out(ob, bi):
            @pl.when(bi < _NB1 - 1)
            def _():
                pltpu.async_copy(
                    out_v.at[ob], t2_hbm.at[pl.ds(bi * 64, 64), :], gos[ob])

            @pl.when(bi == _NB1 - 1)
            def _():
                pltpu.async_copy(
                    out_v.at[ob, pl.ds(0, 32)],
                    t2_hbm.at[pl.ds(bi * 64, 32), :], gos[ob])

        def wait_out(ob, bi):
            @pl.when(bi < _NB1 - 1)
            def _():
                pltpu.make_async_copy(
                    out_v.at[ob], t2_hbm.at[pl.ds(0, 64), :], gos[ob]).wait()

            @pl.when(bi == _NB1 - 1)
            def _():
                pltpu.make_async_copy(
                    out_v.at[ob, pl.ds(0, 32)],
                    t2_hbm.at[pl.ds(0, 32), :], gos[ob]).wait()

        def transpose(j, ob):
            @plsc.parallel_loop(0, 64, 1, unroll=4)
            def _t(p):
                for h in range(2):
                    col = jnp.full((16,), 2, jnp.int32) * p + h
                    for g in range(4):
                        vec = plsc.load_gather(
                            in_v.at[j], [g * 16 + rows16, col])
                        out_v[ob, p, pl.ds(h * 64 + g * 16, 16)] = vec

        for j in range(4):
            start_in(j, wid + j * _NW)

        def body(s, carry):
            for j in range(4):
                m = 4 * s + j
                bi = wid + m * _NW

                @pl.when(bi < _NB1)
                def _sec(j=j, m=m, bi=bi):
                    wait_in(j)

                    @pl.when(m >= 2)
                    def _():
                        wait_out(j % 2, bi - 2 * _NW)

                    transpose(j, j % 2)
                    start_out(j % 2, bi)

                    @pl.when(bi + 4 * _NW < _NB1)
                    def _():
                        start_in(j, bi + 4 * _NW)

            return carry

        lax.fori_loop(0, _S1, body, 0)

        nblk = (_NB1 - 1 - wid) // _NW + 1
        last_even = ((nblk - 1) // 2) * 2
        last_odd = ((nblk - 2) // 2) * 2 + 1
        wait_out(0, wid + last_even * _NW)
        wait_out(1, wid + last_odd * _NW)

    return retile_kernel


def _make_gather():
    mesh = plsc.VectorSubcoreMesh(core_axis_name="c", subcore_axis_name="s")

    @functools.partial(
        pl.kernel,
        mesh=mesh,
        out_type=jax.ShapeDtypeStruct((SEQ_LEN, EMBEDDING_DIM, BATCH),
                                      jnp.float32),
        compiler_params=_params,
        scratch_types=[
            pltpu.VMEM((4, _CB), jnp.int32),
            pltpu.VMEM((_CB,), jnp.int32),
            pltpu.VMEM((_CB,), jnp.int32),
            pltpu.VMEM((_CB,), jnp.int32),
            pltpu.VMEM((_CB,), jnp.int32),
            pltpu.VMEM((4, _CB, 128), jnp.float32),
            pltpu.VMEM((2, EMBEDDING_DIM, _CB), jnp.float32),
            pltpu.SemaphoreType.DMA,
            pltpu.SemaphoreType.DMA,
            pltpu.SemaphoreType.DMA,
            pltpu.SemaphoreType.DMA,
            pltpu.SemaphoreType.DMA,
            pltpu.SemaphoreType.DMA,
            pltpu.SemaphoreType.DMA,
            pltpu.SemaphoreType.DMA,
            pltpu.SemaphoreType.DMA,
            pltpu.SemaphoreType.DMA,
        ],
    )
    def gather_kernel(t2_hbm, idx_hbm, out_hbm, idx_v, px0, px1, px2, px3,
                      g_v, o_v, gg0, gg1, gg2, gg3, gx0, gx1, gx2, gx3,
                      go0, go1):
        wid = lax.axis_index("s") * _NC + lax.axis_index("c")
        rows16 = _iota16()
        ggs = (gg0, gg1, gg2, gg3)
        gxs = (gx0, gx1, gx2, gx3)
        gos = (go0, go1)
        pxs = (px0, px1, px2, px3)
        nbt = BATCH // _CB

        def uaddr(m):
            u = wid + m * _NW
            return u // nbt, (u % nbt) * _CB

        def start_idx(j, m):
            t, b0 = uaddr(m)
            pltpu.async_copy(
                idx_hbm.at[t, pl.ds(b0, _CB)], idx_v.at[j], gxs[j])

        def wait_idx_make_pidx(j):
            pltpu.make_async_copy(
                idx_hbm.at[0, pl.ds(0, _CB)], idx_v.at[j], gxs[j]).wait()
            for ig in range(_CB // 16):
                pxs[j][pl.ds(ig * 16, 16)] = idx_v[j, pl.ds(ig * 16, 16)] >> 1

        def start_gather(j):
            pltpu.async_copy(t2_hbm.at[pxs[j]], g_v.at[j], ggs[j])

        def wait_gather(j):
            pltpu.make_async_copy(
                t2_hbm.at[pxs[j]], g_v.at[j], ggs[j]).wait()

        def start_out(ob, m):
            t, b0 = uaddr(m)
            pltpu.async_copy(
                o_v.at[ob], out_hbm.at[t, :, pl.ds(b0, _CB)], gos[ob])

        def wait_out(ob):
            pltpu.make_async_copy(
                o_v.at[ob], out_hbm.at[0, :, pl.ds(0, _CB)], gos[ob]).wait()

        def transpose(j, ob):
            @plsc.parallel_loop(0, _CB // 16, 1, unroll=2)
            def _t(ig):
                halfv = (idx_v[j, pl.ds(ig * 16, 16)] & 1) * 64
                rows = ig * 16 + rows16
                for d in range(EMBEDDING_DIM):
                    vec = plsc.load_gather(g_v.at[j], [rows, halfv + d])
                    o_v[ob, d, pl.ds(ig * 16, 16)] = vec

        for j in range(4):
            start_idx(j, j)
        wait_idx_make_pidx(0)
        start_gather(0)
        wait_idx_make_pidx(1)
        start_gather(1)

        def body(s, carry):
            for j in range(4):
                m = 4 * s + j
                jn = (j + 2) % 4

                wait_gather(j)

                @pl.when(m >= 2)
                def _(j=j):
                    wait_out(j % 2)

                transpose(j, j % 2)
                start_out(j % 2, m)

                @pl.when(m + 4 < _NUPW)
                def _(j=j, m=m):
                    start_idx(j, m + 4)

                @pl.when(m + 2 < _NUPW)
                def _(jn=jn, m=m):
                    wait_idx_make_pidx(jn)
                    start_gather(jn)

            return carry

        lax.fori_loop(0, _NUPW // 4, body, 0)
        wait_out(0)
        wait_out(1)

    return gather_kernel


_retile = _make_retile()
_gather = _make_gather()


def kernel(input, weight):
    t2 = _retile(weight.T)
    out_t = _gather(t2, input)
    return jnp.transpose(out_t, (0, 2, 1))
